# baseline (speedup 1.0000x reference)
import functools
import math

import jax
import jax.numpy as jnp
from jax import lax
from jax.experimental import pallas as pl
from jax.experimental.pallas import tpu as pltpu
from jax.experimental.pallas import tpu_sc as plsc

D_MODEL = 128
SCALE = math.sqrt(float(D_MODEL))
NUM_CORES = 2
NUM_SUBCORES = 16
NUM_WORKERS = NUM_CORES * NUM_SUBCORES
LANES = 16
CHUNK = 128


def _scale_chunk(buf):
    def row_body(i, _):
        for j in range(D_MODEL // LANES):
            sl = pl.ds(j * LANES, LANES)
            buf[i, sl] = buf[i, sl] * SCALE
        return 0

    lax.fori_loop(0, CHUNK, row_body, 0)


@functools.partial(jax.jit, static_argnames=("n_chunks",))
def _embed_sc(x2d, table, n_chunks):
    rows_total = NUM_WORKERS * n_chunks * CHUNK
    mesh = plsc.VectorSubcoreMesh(core_axis_name="c", subcore_axis_name="s")
    NBUF = 4

    @functools.partial(
        pl.kernel,
        mesh=mesh,
        out_type=jax.ShapeDtypeStruct((rows_total, D_MODEL), jnp.float32),
        scratch_types=[
            pltpu.VMEM((n_chunks, CHUNK), jnp.int32),
        ]
        + [pltpu.VMEM((CHUNK, D_MODEL), jnp.float32)] * NBUF
        + [pltpu.SemaphoreType.DMA] * (2 * NBUF),
    )
    def k(x_hbm, table_hbm, out_hbm, idx_v, *bufs_sems):
        bufs = bufs_sems[:NBUF]
        gsems = bufs_sems[NBUF : 2 * NBUF]
        ssems = bufs_sems[2 * NBUF :]
        wid = lax.axis_index("s") * NUM_CORES + lax.axis_index("c")
        row_base = wid * (n_chunks * CHUNK)

        pltpu.sync_copy(x_hbm.at[pl.ds(wid * n_chunks, n_chunks)], idx_v)

        def start_gather(g, p):
            pltpu.make_async_copy(
                table_hbm.at[idx_v.at[g]], bufs[p], gsems[p]
            ).start()

        def wait_gather(p):
            pltpu.make_async_copy(
                table_hbm.at[pl.ds(0, CHUNK)], bufs[p], gsems[p]
            ).wait()

        def start_scatter(g, p):
            pltpu.make_async_copy(
                bufs[p], out_hbm.at[pl.ds(row_base + g * CHUNK, CHUNK)], ssems[p]
            ).start()

        def wait_scatter(p):
            pltpu.make_async_copy(
                bufs[p], out_hbm.at[pl.ds(row_base, CHUNK)], ssems[p]
            ).wait()

        start_gather(0, 0)
        start_gather(1, 1)

        def loop_body(gg, _):
            for b in range(NBUF):
                g = gg + b
                p = b
                pf = (b + 2) % NBUF
                if b < 2:
                    @pl.when(gg > 0)
                    def _():
                        wait_scatter(pf)

                    start_gather(g + 2, pf)
                else:
                    @pl.when(gg < n_chunks - NBUF)
                    def _():
                        wait_scatter(pf)
                        start_gather(g + 2, pf)

                wait_gather(p)
                _scale_chunk(bufs[p])
                start_scatter(g, p)
            return 0

        lax.fori_loop(0, n_chunks // NBUF, lambda t, c: loop_body(t * NBUF, c), 0)

        for p in range(NBUF):
            wait_scatter(p)

    return k(x2d, table)


def kernel(x, table):
    seq_shape = x.shape
    n_idx = x.size
    assert n_idx % (NUM_WORKERS * CHUNK) == 0
    n_chunks = n_idx // (NUM_WORKERS * CHUNK)
    x2d = jnp.reshape(x.astype(jnp.int32), (NUM_WORKERS * n_chunks, CHUNK))
    out = _embed_sc(x2d, table, n_chunks)
    return jnp.reshape(out, seq_shape + (D_MODEL,))


# speedup vs baseline: 1.0045x; 1.0045x over previous
import functools
import math

import jax
import jax.numpy as jnp
from jax import lax
from jax.experimental import pallas as pl
from jax.experimental.pallas import tpu as pltpu
from jax.experimental.pallas import tpu_sc as plsc

D_MODEL = 128
SCALE = math.sqrt(float(D_MODEL))
NUM_CORES = 2
NUM_SUBCORES = 16
NUM_WORKERS = NUM_CORES * NUM_SUBCORES
LANES = 16
CHUNK = 128


def _scale_chunk(buf):
    def row_body(i, _):
        for j in range(D_MODEL // LANES):
            sl = pl.ds(j * LANES, LANES)
            buf[i, sl] = buf[i, sl] * SCALE
        return 0

    lax.fori_loop(0, CHUNK, row_body, 0)


@functools.partial(jax.jit, static_argnames=("n_chunks",))
def _embed_sc(x2d, table, n_chunks):
    rows_total = NUM_WORKERS * n_chunks * CHUNK
    mesh = plsc.VectorSubcoreMesh(core_axis_name="c", subcore_axis_name="s")
    NBUF = 5
    PD = 2

    @functools.partial(
        pl.kernel,
        mesh=mesh,
        out_type=jax.ShapeDtypeStruct((rows_total, D_MODEL), jnp.float32),
        scratch_types=[
            pltpu.VMEM((n_chunks, CHUNK), jnp.int32),
        ]
        + [pltpu.VMEM((CHUNK, D_MODEL), jnp.float32)] * NBUF
        + [pltpu.SemaphoreType.DMA] * (2 * NBUF),
    )
    def k(x_hbm, table_hbm, out_hbm, idx_v, *bufs_sems):
        bufs = bufs_sems[:NBUF]
        gsems = bufs_sems[NBUF : 2 * NBUF]
        ssems = bufs_sems[2 * NBUF :]
        wid = lax.axis_index("s") * NUM_CORES + lax.axis_index("c")
        row_base = wid * (n_chunks * CHUNK)

        pltpu.sync_copy(x_hbm.at[pl.ds(wid * n_chunks, n_chunks)], idx_v)

        def start_gather(g, p):
            pltpu.make_async_copy(
                table_hbm.at[idx_v.at[g]], bufs[p], gsems[p]
            ).start()

        def wait_gather(p):
            pltpu.make_async_copy(
                table_hbm.at[pl.ds(0, CHUNK)], bufs[p], gsems[p]
            ).wait()

        def start_scatter(g, p):
            pltpu.make_async_copy(
                bufs[p], out_hbm.at[pl.ds(row_base + g * CHUNK, CHUNK)], ssems[p]
            ).start()

        def wait_scatter(p):
            pltpu.make_async_copy(
                bufs[p], out_hbm.at[pl.ds(row_base, CHUNK)], ssems[p]
            ).wait()

        start_gather(0, 0)
        start_gather(1, 1)

        def loop_body(gg, _):
            for b in range(NBUF):
                g = gg + b
                p = b
                pf = (b + PD) % NBUF
                if b < NBUF - PD:
                    @pl.when(gg > 0)
                    def _():
                        wait_scatter(pf)

                    start_gather(g + PD, pf)
                else:
                    @pl.when(gg < n_chunks - NBUF)
                    def _():
                        wait_scatter(pf)
                        start_gather(g + PD, pf)

                wait_gather(p)
                _scale_chunk(bufs[p])
                start_scatter(g, p)
            return 0

        lax.fori_loop(0, n_chunks // NBUF, lambda t, c: loop_body(t * NBUF, c), 0)

        for p in range(NBUF):
            wait_scatter(p)

    return k(x2d, table)


def kernel(x, table):
    seq_shape = x.shape
    n_idx = x.size
    assert n_idx % (NUM_WORKERS * CHUNK) == 0
    n_chunks = n_idx // (NUM_WORKERS * CHUNK)
    x2d = jnp.reshape(x.astype(jnp.int32), (NUM_WORKERS * n_chunks, CHUNK))
    out = _embed_sc(x2d, table, n_chunks)
    return jnp.reshape(out, seq_shape + (D_MODEL,))


# speedup vs baseline: 1.0237x; 1.0192x over previous
import functools
import math

import jax
import jax.numpy as jnp
from jax import lax
from jax.experimental import pallas as pl
from jax.experimental.pallas import tpu as pltpu
from jax.experimental.pallas import tpu_sc as plsc

D_MODEL = 128
SCALE = math.sqrt(float(D_MODEL))
NUM_CORES = 2
NUM_SUBCORES = 16
NUM_WORKERS = NUM_CORES * NUM_SUBCORES
LANES = 16
CHUNK = 64


def _scale_chunk(buf):
    def row_body(i, _):
        for j in range(D_MODEL // LANES):
            sl = pl.ds(j * LANES, LANES)
            buf[i, sl] = buf[i, sl] * SCALE
        return 0

    lax.fori_loop(0, CHUNK, row_body, 0)


@functools.partial(jax.jit, static_argnames=("n_chunks",))
def _embed_sc(x2d, table, n_chunks):
    rows_total = NUM_WORKERS * n_chunks * CHUNK
    mesh = plsc.VectorSubcoreMesh(core_axis_name="c", subcore_axis_name="s")
    NBUF = 4
    PD = 2

    @functools.partial(
        pl.kernel,
        mesh=mesh,
        out_type=jax.ShapeDtypeStruct((rows_total, D_MODEL), jnp.float32),
        scratch_types=[
            pltpu.VMEM((n_chunks, CHUNK), jnp.int32),
            pltpu.VMEM_SHARED((NUM_SUBCORES, NBUF, CHUNK, D_MODEL), jnp.float32),
        ]
        + [pltpu.VMEM((CHUNK, D_MODEL), jnp.float32)] * NBUF
        + [pltpu.SemaphoreType.DMA] * (3 * NBUF),
    )
    def k(x_hbm, table_hbm, out_hbm, idx_v, shared, *bufs_sems):
        bufs = bufs_sems[:NBUF]
        gsems = bufs_sems[NBUF : 2 * NBUF]
        c1sems = bufs_sems[2 * NBUF : 3 * NBUF]
        c2sems = bufs_sems[3 * NBUF :]
        sid = lax.axis_index("s")
        wid = sid * NUM_CORES + lax.axis_index("c")
        row_base = wid * (n_chunks * CHUNK)

        pltpu.sync_copy(x_hbm.at[pl.ds(wid * n_chunks, n_chunks)], idx_v)

        def start_gather(g, p):
            pltpu.make_async_copy(
                table_hbm.at[idx_v.at[g]], bufs[p], gsems[p]
            ).start()

        def wait_gather(p):
            pltpu.make_async_copy(
                table_hbm.at[pl.ds(0, CHUNK)], bufs[p], gsems[p]
            ).wait()

        def start_spmem_put(p):
            pltpu.make_async_copy(bufs[p], shared.at[sid, p], c1sems[p]).start()

        def wait_spmem_put(p):
            pltpu.make_async_copy(bufs[p], shared.at[sid, p], c1sems[p]).wait()

        def start_out_write(g, q):
            pltpu.make_async_copy(
                shared.at[sid, q],
                out_hbm.at[pl.ds(row_base + g * CHUNK, CHUNK)],
                c2sems[q],
            ).start()

        def wait_out_write(q):
            pltpu.make_async_copy(
                shared.at[sid, q],
                out_hbm.at[pl.ds(row_base, CHUNK)],
                c2sems[q],
            ).wait()

        start_gather(0, 0)
        start_gather(1, 1)

        def loop_body(gg, _):
            for b in range(NBUF):
                g = gg + b
                p = b
                pf = (b + PD) % NBUF
                if b < NBUF - PD:
                    start_gather(g + PD, pf)
                else:
                    @pl.when(gg < n_chunks - NBUF)
                    def _():
                        start_gather(g + PD, pf)

                wait_gather(p)
                _scale_chunk(bufs[p])

                @pl.when(gg > 0)
                def _():
                    wait_out_write(p)

                start_spmem_put(p)

                qprev = (b - 1) % NBUF
                if b == 0:
                    @pl.when(gg > 0)
                    def _():
                        wait_spmem_put(qprev)
                        start_out_write(gg - 1, qprev)
                else:
                    wait_spmem_put(qprev)
                    start_out_write(g - 1, qprev)
            return 0

        lax.fori_loop(0, n_chunks // NBUF, lambda t, c: loop_body(t * NBUF, c), 0)

        wait_spmem_put(NBUF - 1)
        start_out_write(n_chunks - 1, NBUF - 1)
        for q in range(NBUF):
            wait_out_write(q)

    return k(x2d, table)


def kernel(x, table):
    seq_shape = x.shape
    n_idx = x.size
    assert n_idx % (NUM_WORKERS * CHUNK) == 0
    n_chunks = n_idx // (NUM_WORKERS * CHUNK)
    x2d = jnp.reshape(x.astype(jnp.int32), (NUM_WORKERS * n_chunks, CHUNK))
    out = _embed_sc(x2d, table, n_chunks)
    return jnp.reshape(out, seq_shape + (D_MODEL,))
